# Initial kernel scaffold; baseline (speedup 1.0000x reference)
import jax
import jax.numpy as jnp
from jax.experimental import pallas as pl


def kernel(x, edge_index, Ws, bs):
    raise NotImplementedError("write your pallas kernel here")


# speedup vs baseline: 5.1710x; 5.1710x over previous
import functools

import jax
import jax.numpy as jnp
from jax import lax
from jax.experimental import pallas as pl
from jax.experimental.pallas import tpu as pltpu
from jax.experimental.pallas import tpu_sc as plsc

N = 10000
E = 320000
NUM_CORES = 2
NUM_SUBCORES = 16
NW = NUM_CORES * NUM_SUBCORES
EPW = E // NW
BATCH = 80
NBATCH = EPW // BATCH
ROWS_PER_SUB = 624
ROWS_TAIL = N - ROWS_PER_SUB * NUM_SUBCORES

LAYER_DIMS = [(128, 128), (128, 192), (192, 256), (256, 256), (256, 256),
              (256, 256), (256, 128), (128, 192), (192, 1)]
LAYER_RELU = [True, True, True, True, False, True, False, False, False]
LAYER_PRE = [din <= dout for din, dout in LAYER_DIMS]

ROW_BLK = 1000


def _chunks(d):
    if d == 1:
        return [16]
    out = [128] * (d // 128)
    if d % 128:
        out.append(d % 128)
    return out


@functools.cache
def _sc_propagate(dc):
    mesh = plsc.VectorSubcoreMesh(core_axis_name="c", subcore_axis_name="s")

    def body(u_hbm, src_hbm, dst_hbm, out_hbm, src_idx, dst_idx, rows, acc):
        cid = lax.axis_index("c")
        sid = lax.axis_index("s")
        wid = sid * NUM_CORES + cid
        r0 = sid * ROWS_PER_SUB
        pltpu.sync_copy(u_hbm.at[pl.ds(r0, ROWS_PER_SUB), :],
                        acc.at[pl.ds(r0, ROWS_PER_SUB), :])

        @pl.when(sid == NUM_SUBCORES - 1)
        def _():
            t0 = ROWS_PER_SUB * NUM_SUBCORES
            pltpu.sync_copy(u_hbm.at[pl.ds(t0, ROWS_TAIL), :],
                            acc.at[pl.ds(t0, ROWS_TAIL), :])

        plsc.subcore_barrier()
        base = wid * EPW

        @pl.loop(0, NBATCH)
        def _(b):
            off = pl.multiple_of(base + b * BATCH, 8)
            pltpu.sync_copy(src_hbm.at[pl.ds(off, BATCH)], src_idx)
            pltpu.sync_copy(dst_hbm.at[pl.ds(off, BATCH)], dst_idx)
            pltpu.sync_copy(u_hbm.at[src_idx], rows)
            pltpu.sync_copy(rows, acc.at[dst_idx], add=True)

        plsc.subcore_barrier()
        pltpu.sync_copy(acc.at[pl.ds(r0, ROWS_PER_SUB), :],
                        out_hbm.at[cid, pl.ds(r0, ROWS_PER_SUB), :])

        @pl.when(sid == NUM_SUBCORES - 1)
        def _():
            t0 = ROWS_PER_SUB * NUM_SUBCORES
            pltpu.sync_copy(acc.at[pl.ds(t0, ROWS_TAIL), :],
                            out_hbm.at[cid, pl.ds(t0, ROWS_TAIL), :])

    return pl.kernel(
        body,
        out_type=jax.ShapeDtypeStruct((NUM_CORES, N, dc), jnp.float32),
        mesh=mesh,
        compiler_params=pltpu.CompilerParams(use_tc_tiling_on_sc=False),
        scratch_types=[
            pltpu.VMEM((BATCH,), jnp.int32),
            pltpu.VMEM((BATCH,), jnp.int32),
            pltpu.VMEM((BATCH, dc), jnp.float32),
            pltpu.VMEM_SHARED((N, dc), jnp.float32),
        ],
    )


def _tc_inv_sqrt(acc):
    def body(a_ref, o_ref):
        deg = a_ref[0, :, 0:1] + a_ref[1, :, 0:1] - 1.0
        o_ref[...] = lax.rsqrt(deg)

    grid = (N // ROW_BLK,)
    return pl.pallas_call(
        body,
        out_shape=jax.ShapeDtypeStruct((N, 1), jnp.float32),
        grid=grid,
        in_specs=[pl.BlockSpec((2, ROW_BLK, 16), lambda i: (0, i, 0))],
        out_specs=pl.BlockSpec((ROW_BLK, 1), lambda i: (i, 0)),
    )(acc)


def _tc_scale_chunks(h, inv, chunks, pad_single_col):
    d = h.shape[1]

    def body(h_ref, i_ref, *o_refs):
        u = h_ref[...] * i_ref[...]
        if pad_single_col:
            o_refs[0][...] = jnp.pad(u, ((0, 0), (0, 15)))
        else:
            c0 = 0
            for o_ref, dc in zip(o_refs, chunks):
                o_ref[...] = u[:, c0:c0 + dc]
                c0 += dc

    grid = (N // ROW_BLK,)
    return pl.pallas_call(
        body,
        out_shape=[jax.ShapeDtypeStruct((N, dc), jnp.float32) for dc in chunks],
        grid=grid,
        in_specs=[pl.BlockSpec((ROW_BLK, d), lambda i: (i, 0)),
                  pl.BlockSpec((ROW_BLK, 1), lambda i: (i, 0))],
        out_specs=[pl.BlockSpec((ROW_BLK, dc), lambda i: (i, 0))
                   for dc in chunks],
    )(h, inv)


def _tc_combine_matmul(accs, us, inv, w, b, relu):
    d, dout = w.shape
    chunks = [u.shape[1] for u in us]
    nch = len(chunks)

    def body(*refs):
        a_refs = refs[0:nch]
        u_refs = refs[nch:2 * nch]
        i_ref, w_ref, b_ref, o_ref = refs[2 * nch:]
        ps = []
        for a_ref, u_ref in zip(a_refs, u_refs):
            ps.append(i_ref[...] * (a_ref[0] + a_ref[1] - u_ref[...]))
        p = ps[0] if nch == 1 else jnp.concatenate(ps, axis=1)
        out = jnp.dot(p, w_ref[...], preferred_element_type=jnp.float32)
        out = out + b_ref[...]
        o_ref[...] = jnp.maximum(out, 0.0) if relu else out

    grid = (N // ROW_BLK,)
    in_specs = (
        [pl.BlockSpec((2, ROW_BLK, dc), lambda i: (0, i, 0)) for dc in chunks]
        + [pl.BlockSpec((ROW_BLK, dc), lambda i: (i, 0)) for dc in chunks]
        + [pl.BlockSpec((ROW_BLK, 1), lambda i: (i, 0)),
           pl.BlockSpec((d, dout), lambda i: (0, 0)),
           pl.BlockSpec((1, dout), lambda i: (0, 0))]
    )
    return pl.pallas_call(
        body,
        out_shape=jax.ShapeDtypeStruct((N, dout), jnp.float32),
        grid=grid,
        in_specs=in_specs,
        out_specs=pl.BlockSpec((ROW_BLK, dout), lambda i: (i, 0)),
    )(*accs, *us, inv, w, b.reshape(1, dout))


def _tc_matmul(h, w):
    d, dout = w.shape

    def body(h_ref, w_ref, o_ref):
        o_ref[...] = jnp.dot(h_ref[...], w_ref[...],
                             preferred_element_type=jnp.float32)

    grid = (N // ROW_BLK,)
    return pl.pallas_call(
        body,
        out_shape=jax.ShapeDtypeStruct((N, dout), jnp.float32),
        grid=grid,
        in_specs=[pl.BlockSpec((ROW_BLK, d), lambda i: (i, 0)),
                  pl.BlockSpec((d, dout), lambda i: (0, 0))],
        out_specs=pl.BlockSpec((ROW_BLK, dout), lambda i: (i, 0)),
    )(h, w)


def _tc_combine_elem(acc, u, inv, b, relu, dout):
    dc = u.shape[1]

    def body(a_ref, u_ref, i_ref, b_ref, o_ref):
        p = i_ref[...] * (a_ref[0] + a_ref[1] - u_ref[...])
        out = p[:, 0:dout] + b_ref[...]
        o_ref[...] = jnp.maximum(out, 0.0) if relu else out

    grid = (N // ROW_BLK,)
    return pl.pallas_call(
        body,
        out_shape=jax.ShapeDtypeStruct((N, dout), jnp.float32),
        grid=grid,
        in_specs=[pl.BlockSpec((2, ROW_BLK, dc), lambda i: (0, i, 0)),
                  pl.BlockSpec((ROW_BLK, dc), lambda i: (i, 0)),
                  pl.BlockSpec((ROW_BLK, 1), lambda i: (i, 0)),
                  pl.BlockSpec((1, dout), lambda i: (0, 0))],
        out_specs=pl.BlockSpec((ROW_BLK, dout), lambda i: (i, 0)),
    )(acc, u, inv, b.reshape(1, dout))


def _propagate(v, inv, src, dst):
    d = v.shape[1]
    chunks = _chunks(d)
    us = _tc_scale_chunks(v, inv, chunks, pad_single_col=(d == 1))
    accs = [_sc_propagate(dc)(u, src, dst) for dc, u in zip(chunks, us)]
    return accs, us


def kernel(x, edge_index, Ws, bs):
    src = edge_index[0]
    dst = edge_index[1]

    ones16 = jnp.ones((N, 16), jnp.float32)
    deg_acc = _sc_propagate(16)(ones16, src, dst)
    inv = _tc_inv_sqrt(deg_acc)

    h = x
    for i, ((din, dout), relu, pre) in enumerate(
            zip(LAYER_DIMS, LAYER_RELU, LAYER_PRE)):
        w, b = Ws[i], bs[i]
        if pre:
            accs, us = _propagate(h, inv, src, dst)
            h = _tc_combine_matmul(accs, us, inv, w, b, relu)
        else:
            q = _tc_matmul(h, w)
            accs, us = _propagate(q, inv, src, dst)
            h = _tc_combine_elem(accs[0], us[0], inv, b, relu, dout)
    return h
